# baseline (speedup 1.0000x reference)
import functools
import math

import jax
import jax.numpy as jnp
from jax import lax
from jax.experimental import pallas as pl
from jax.experimental.pallas import tpu as pltpu
from jax.experimental.pallas import tpu_sc as plsc

_DT_SCALE = math.log1p(168.0)

_B, _L = 16384, 200
_NC, _NS, _LANES = 2, 16, 16
_NW = _NC * _NS
_BR = 8
_BC = _B // _NW
_TC_ROWS = 144
_SC_R0 = _TC_ROWS
_NR = (_L - _TC_ROWS) // _BR

_mesh = plsc.VectorSubcoreMesh(core_axis_name="c", subcore_axis_name="s")

_GATHER_DNUMS = lax.GatherDimensionNumbers(
    offset_dims=(), collapsed_slice_dims=(0,), start_index_map=(0,))


@functools.partial(
    pl.kernel,
    mesh=_mesh,
    out_type=jax.ShapeDtypeStruct((_L - _SC_R0, _B), jnp.float32),
    scratch_types=[
        pltpu.VMEM((_LANES,), jnp.float32),
        pltpu.VMEM((2, _BR, _BC), jnp.float32),
        pltpu.VMEM((2, _BR, _BC), jnp.float32),
        pltpu.VMEM((2, _BR, _BC), jnp.int32),
        pltpu.VMEM((2, _BR, _BC), jnp.float32),
        pltpu.SemaphoreType.DMA,
        pltpu.SemaphoreType.DMA,
        pltpu.SemaphoreType.DMA,
        pltpu.SemaphoreType.DMA,
    ],
    compiler_params=pltpu.CompilerParams(
        use_tc_tiling_on_sc=True, skip_device_barrier=True,
        disable_bounds_checks=True),
)
def _recovery_sc(mpc_hbm, dt_hbm, idx_hbm, tab_hbm, out_hbm,
                 tab_v, mpc_v, dt_v, idx_v, out_v,
                 in_sem0, in_sem1, out_sem0, out_sem1):
    wid = lax.axis_index("s") * _NC + lax.axis_index("c")
    c0 = wid * _BC
    in_sems = (in_sem0, in_sem1)
    out_sems = (out_sem0, out_sem1)

    pltpu.sync_copy(tab_hbm, tab_v)
    tab_vec = -jnp.exp(-tab_v[...])

    def in_copies(k, b):
        blk = (pl.ds(_SC_R0 + k * _BR, _BR), pl.ds(c0, _BC))
        return (
            pltpu.make_async_copy(mpc_hbm.at[blk], mpc_v.at[b], in_sems[b]),
            pltpu.make_async_copy(dt_hbm.at[blk], dt_v.at[b], in_sems[b]),
            pltpu.make_async_copy(idx_hbm.at[blk], idx_v.at[b], in_sems[b]),
        )

    def out_copy(k, b):
        blk = (pl.ds(k * _BR, _BR), pl.ds(c0, _BC))
        return pltpu.make_async_copy(out_v.at[b], out_hbm.at[blk], out_sems[b])

    def start_in(k, b):
        for c in in_copies(k, b):
            c.start()

    def compute(b):
        @plsc.parallel_loop(0, _BC, _LANES, unroll=4)
        def body(c):
            for r in range(_BR):
                sl = pl.ds(c, _LANES)
                neg_inv_tau = lax.gather(
                    tab_vec, idx_v[b, r, sl][:, None], _GATHER_DNUMS, (1,),
                    mode=lax.GatherScatterMode.PROMISE_IN_BOUNDS)
                dt_hours = jnp.exp(dt_v[b, r, sl] * _DT_SCALE) - 1.0
                decay = jnp.exp(dt_hours * neg_inv_tau)
                out_v[b, r, sl] = 1.0 - (1.0 - mpc_v[b, r, sl]) * decay

    start_in(0, 0)

    def round_pair(k, _):
        for b in range(2):
            kb = k + b

            @pl.when(kb < _NR)
            def _():
                @pl.when(kb + 1 < _NR)
                def _():
                    start_in(kb + 1, 1 - b)

                for c in in_copies(kb, b):
                    c.wait()

                @pl.when(kb >= 2)
                def _():
                    out_copy(kb - 2, b).wait()

                compute(b)
                out_copy(kb, b).start()
        return 0

    lax.fori_loop(0, (_NR + 1) // 2, lambda k, s: round_pair(2 * k, s), 0)

    out_copy(_NR - 2, (_NR - 2) % 2).wait()
    out_copy(_NR - 1, (_NR - 1) % 2).wait()


_TBC = 4096


def _tc_body(mpc_ref, dt_ref, idx_ref, tab_ref, out_ref):
    tab = [-jnp.exp(-tab_ref[j]) for j in range(15)]
    idx = idx_ref[...]
    nit = jnp.full((_BR, _TBC), tab[0], jnp.float32)
    for j in range(1, 15):
        nit = jnp.where(idx == j, tab[j], nit)
    dt_hours = jnp.exp(dt_ref[...] * _DT_SCALE) - 1.0
    decay = jnp.exp(dt_hours * nit)
    out_ref[...] = 1.0 - (1.0 - mpc_ref[...]) * decay


_in_spec = pl.BlockSpec((_BR, _TBC), lambda i, j: (i, j))

_recovery_tc = pl.pallas_call(
    _tc_body,
    grid=(_TC_ROWS // _BR, _B // _TBC),
    in_specs=[_in_spec, _in_spec, _in_spec,
              pl.BlockSpec(memory_space=pltpu.SMEM)],
    out_specs=pl.BlockSpec((_BR, _TBC), lambda i, j: (i, j)),
    out_shape=jax.ShapeDtypeStruct((_L, _B), jnp.float32),
)


def _merge_body(full_ref, sc_ref, out_ref):
    out_ref[...] = sc_ref[...]


_merge = pl.pallas_call(
    _merge_body,
    grid=((_L - _SC_R0) // _BR,),
    in_specs=[pl.BlockSpec(memory_space=pltpu.MemorySpace.HBM),
              pl.BlockSpec((_BR, _B), lambda i: (i, 0))],
    out_specs=pl.BlockSpec((_BR, _B), lambda i: (i + _SC_R0 // _BR, 0)),
    out_shape=jax.ShapeDtypeStruct((_L, _B), jnp.float32),
    input_output_aliases={0: 0},
)


def kernel(mpc, delta_t, muscle_idx, log_tau):
    idx = muscle_idx.astype(jnp.int32)
    tab = jnp.pad(log_tau.astype(jnp.float32), (0, _LANES - log_tau.shape[0]))
    mpc_t, dt_t, idx_t = mpc.T, delta_t.T, idx.T
    sc_part = _recovery_sc(mpc_t, dt_t, idx_t, tab)
    tc_full = _recovery_tc(mpc_t, dt_t, idx_t, tab)
    merged = _merge(tc_full, sc_part)
    return merged.T


# speedup vs baseline: 1.4131x; 1.4131x over previous
import functools
import math

import jax
import jax.numpy as jnp
from jax import lax
from jax.experimental import pallas as pl
from jax.experimental.pallas import tpu as pltpu
from jax.experimental.pallas import tpu_sc as plsc

_DT_SCALE = math.log1p(168.0)
_LOG2E = math.log2(math.e)
_DT_SCALE2 = _DT_SCALE * _LOG2E

_B, _L = 16384, 200
_NC, _NS, _LANES = 2, 16, 16
_NW = _NC * _NS
_BR = 8
_BC = _B // _NW
_NR = _L // _BR

_mesh = plsc.VectorSubcoreMesh(core_axis_name="c", subcore_axis_name="s")

_GATHER_DNUMS = lax.GatherDimensionNumbers(
    offset_dims=(), collapsed_slice_dims=(0,), start_index_map=(0,))


@functools.partial(
    pl.kernel,
    mesh=_mesh,
    out_type=jax.ShapeDtypeStruct((_L, _B), jnp.float32),
    scratch_types=[
        pltpu.VMEM((_LANES,), jnp.float32),
        pltpu.VMEM((2, _BR, _BC), jnp.float32),
        pltpu.VMEM((2, _BR, _BC), jnp.float32),
        pltpu.VMEM((2, _BR, _BC), jnp.int32),
        pltpu.VMEM((2, _BR, _BC), jnp.float32),
        pltpu.SemaphoreType.DMA,
        pltpu.SemaphoreType.DMA,
        pltpu.SemaphoreType.DMA,
        pltpu.SemaphoreType.DMA,
    ],
    compiler_params=pltpu.CompilerParams(use_tc_tiling_on_sc=True, skip_device_barrier=True, disable_bounds_checks=True),
)
def _recovery(mpc_hbm, dt_hbm, idx_hbm, tab_hbm, out_hbm,
              tab_v, mpc_v, dt_v, idx_v, out_v,
              in_sem0, in_sem1, out_sem0, out_sem1):
    wid = lax.axis_index("s") * _NC + lax.axis_index("c")
    c0 = wid * _BC
    in_sems = (in_sem0, in_sem1)
    out_sems = (out_sem0, out_sem1)

    pltpu.sync_copy(tab_hbm, tab_v)
    tab_vec = -jnp.exp(-tab_v[...])

    def in_copies(k, b):
        r0 = k * _BR
        blk = (pl.ds(r0, _BR), pl.ds(c0, _BC))
        return (
            pltpu.make_async_copy(mpc_hbm.at[blk], mpc_v.at[b], in_sems[b]),
            pltpu.make_async_copy(dt_hbm.at[blk], dt_v.at[b], in_sems[b]),
            pltpu.make_async_copy(idx_hbm.at[blk], idx_v.at[b], in_sems[b]),
        )

    def out_copy(k, b):
        r0 = k * _BR
        blk = (pl.ds(r0, _BR), pl.ds(c0, _BC))
        return pltpu.make_async_copy(out_v.at[b], out_hbm.at[blk], out_sems[b])

    def start_in(k, b):
        for c in in_copies(k, b):
            c.start()

    def compute(b):
        @plsc.parallel_loop(0, _BC, _LANES, unroll=4)
        def body(c):
            for r in range(_BR):
                sl = pl.ds(c, _LANES)
                neg_inv_tau = lax.gather(
                    tab_vec, idx_v[b, r, sl][:, None], _GATHER_DNUMS, (1,),
                    mode=lax.GatherScatterMode.PROMISE_IN_BOUNDS)
                dt_hours = jnp.exp(dt_v[b, r, sl] * _DT_SCALE) - 1.0
                decay = jnp.exp(dt_hours * neg_inv_tau)
                out_v[b, r, sl] = 1.0 - (1.0 - mpc_v[b, r, sl]) * decay

    start_in(0, 0)

    def round_pair(k, _):
        for b in range(2):
            kb = k + b

            @pl.when(kb < _NR)
            def _():
                @pl.when(kb + 1 < _NR)
                def _():
                    start_in(kb + 1, 1 - b)

                for c in in_copies(kb, b):
                    c.wait()

                @pl.when(kb >= 2)
                def _():
                    out_copy(kb - 2, b).wait()

                compute(b)
                out_copy(kb, b).start()
        return 0

    lax.fori_loop(0, (_NR + 1) // 2, lambda k, s: round_pair(2 * k, s), 0)

    out_copy(_NR - 2, (_NR - 2) % 2).wait()
    out_copy(_NR - 1, (_NR - 1) % 2).wait()


def kernel(mpc, delta_t, muscle_idx, log_tau):
    idx = muscle_idx.astype(jnp.int32)
    tab = jnp.pad(log_tau.astype(jnp.float32), (0, _LANES - log_tau.shape[0]))
    out_t = _recovery(mpc.T, delta_t.T, idx.T, tab)
    return out_t.T
